# baseline (speedup 1.0000x reference)
import functools

import numpy as np

import jax
import jax.numpy as jnp
from jax import lax
from jax.experimental import pallas as pl
from jax.experimental.pallas import tpu as pltpu
from jax.experimental.pallas import tpu_sc as plsc

N = 10000
E = 320000
D = 128
H = 2
NEG = 0.2
NS = 16
NC = 2
EPT = E // NS
C = 16
CPS = 25
SLOTS = 5
QPS = CPS // SLOTS
SB = C * CPS
NSB = EPT // SB
GPS = SB // 16
CG = C // 16
CROWS = E // C
RPT = N // NS
WROW = 144


def _nodes_body(x_ref, w_ref, o_ref):
    o_ref[...] = jnp.dot(x_ref[...], w_ref[...],
                         preferred_element_type=jnp.float32)


def _eal_body(ea_ref, w_ref, o_ref):
    acc = jnp.dot(ea_ref[...], w_ref[...], preferred_element_type=jnp.float32)
    o_ref[...] = acc[:, :8]


def _finish_body(zu_ref, w_ref, g_ref, b_ref, o_ref):
    zu = zu_ref[...]
    w = w_ref[...]
    den0 = jnp.maximum(zu[:N, 128:129], 1e-38)
    den1 = jnp.maximum(zu[N:, 128:129], 1e-38)
    z0 = jnp.where(zu[:N, 128:129] > 0, zu[:N, :D] / den0, 0.0)
    z1 = jnp.where(zu[N:, 128:129] > 0, zu[N:, :D] / den1, 0.0)
    ft0 = jnp.dot(z0, w[:, :D], preferred_element_type=jnp.float32)
    ft1 = jnp.dot(z1, w[:, D:], preferred_element_type=jnp.float32)
    h = 0.5 * (ft0 + ft1)
    mu = jnp.mean(h, axis=0, keepdims=True)
    var = jnp.mean((h - mu) ** 2, axis=0, keepdims=True)
    hn = (h - mu) * lax.rsqrt(var + 1e-5) * g_ref[...] + b_ref[...]
    o_ref[...] = jnp.maximum(hn, 0.0)


def _sc_body(x_hbm, ea_hbm, src_hbm, dst2d_hbm, eal_hbm, xlw_hbm,
             erw_hbm, zeros_hbm,
             zu_out,
             zu_sh, sidx, didx2, sadj, dadj, ealb, p16,
             ea2, x2, out2, xlg2, erg2,
             *sems):
    c = lax.axis_index("c")
    s = lax.axis_index("s")
    cn = c * N
    sem_ea = sems[0:SLOTS]
    sem_x = sems[SLOTS:2 * SLOTS]
    sem_l = sems[2 * SLOTS:3 * SLOTS]
    sem_r = sems[3 * SLOTS:4 * SLOTS]
    sem_sc = sems[4 * SLOTS:5 * SLOTS]

    pltpu.sync_copy(zeros_hbm.at[pl.ds(s * RPT, RPT)],
                    zu_sh.at[pl.ds(s * RPT, RPT)])
    plsc.subcore_barrier()

    base = s * EPT
    rowbase = s * (EPT // C)

    def issue(k, j, b):
        e0g = base + k * SB + j * C
        off = j * C
        pltpu.async_copy(ea_hbm.at[pl.ds(e0g, C)], ea2.at[b], sem_ea[b])
        pltpu.async_copy(x_hbm.at[sidx.at[pl.ds(off, C)]], x2.at[b],
                         sem_x[b])
        pltpu.async_copy(xlw_hbm.at[sadj.at[pl.ds(off, C)]], xlg2.at[b],
                         sem_l[b])
        pltpu.async_copy(erw_hbm.at[dadj.at[pl.ds(off, C)]], erg2.at[b],
                         sem_r[b])

    def wait_loads(b):
        pltpu.make_async_copy(ea_hbm.at[pl.ds(0, C)], ea2.at[b],
                              sem_ea[b]).wait()
        pltpu.make_async_copy(x_hbm.at[pl.ds(0, C)], x2.at[b],
                              sem_x[b]).wait()
        pltpu.make_async_copy(xlw_hbm.at[pl.ds(0, C)], xlg2.at[b],
                              sem_l[b]).wait()
        pltpu.make_async_copy(erw_hbm.at[pl.ds(0, C)], erg2.at[b],
                              sem_r[b]).wait()

    def wait_scatter(b, kb):
        pltpu.make_async_copy(out2.at[b], zu_sh.at[didx2.at[kb, 0]],
                              sem_sc[b]).wait()

    def compute(j, b):
        rid = lax.iota(jnp.int32, 16)
        zc = jnp.zeros((16,), jnp.int32)
        xlv = plsc.load_gather(xlg2.at[b], [rid, zc])
        erv = plsc.load_gather(erg2.at[b], [rid, zc])
        att = xlv + ealb[pl.ds(j * C, 16)] + erv
        att = jnp.where(att > 0, att, NEG * att)
        pv = jnp.exp(att)

        for jj in range(C):
            ps = pv[jj]
            for f in range(D // 16):
                out2[b, jj, pl.ds(f * 16, 16)] = (
                    x2[b, jj, pl.ds(f * 16, 16)]
                    + ea2[b, jj, pl.ds(f * 16, 16)]) * ps
            out2[b, jj, pl.ds(D, 16)] = jnp.full((16,), ps, jnp.float32)

    def scatter(j, b, kb):
        pltpu.async_copy(out2.at[b], zu_sh.at[didx2.at[kb, j]],
                         sem_sc[b], add=True)

    def sb_body(k, carry):
        kb = lax.rem(k, 2)
        e0 = base + k * SB
        pltpu.sync_copy(src_hbm.at[pl.ds(e0, SB)], sidx)
        pltpu.sync_copy(dst2d_hbm.at[pl.ds(rowbase + k * CPS, CPS)],
                        didx2.at[kb])
        pltpu.sync_copy(eal_hbm.at[pl.ds(c * E + e0, SB)], ealb)

        def adj_body(g, carry2):
            off = g * 16
            r = lax.div(g, CG)
            co = lax.rem(g, CG) * 16
            sadj[pl.ds(off, 16)] = sidx[pl.ds(off, 16)] + cn
            dadj[pl.ds(off, 16)] = didx2[kb, r, pl.ds(co, 16)] + cn
            return carry2

        lax.fori_loop(0, GPS, adj_body, 0)

        for b in range(SLOTS - 1):
            issue(k, b, b)

        def quint_body(q, carry2):
            j0 = SLOTS * q
            for i in range(SLOTS):
                j = j0 + i
                nb = (i + SLOTS - 1) % SLOTS

                @pl.when(j + SLOTS - 1 < CPS)
                def _(j=j, nb=nb):
                    issue(k, j + SLOTS - 1, nb)

                wait_loads(i)

                @pl.when(jnp.logical_or(k > 0, q >= 1))
                def _(i=i):
                    wait_scatter(i, kb)

                compute(j, i)
                scatter(j, i, kb)
            return carry2

        lax.fori_loop(0, QPS, quint_body, 0)
        return carry

    lax.fori_loop(0, NSB, sb_body, 0)
    for b in range(SLOTS):
        wait_scatter(b, 0)
    plsc.subcore_barrier()
    pltpu.sync_copy(zu_sh.at[pl.ds(s * RPT, RPT)],
                    zu_out.at[pl.ds(c * N + s * RPT, RPT)])


def kernel(x, edge_index, edge_attr, W_fc, attn_l, attn_r, gamma, beta):
    src = edge_index[0].astype(jnp.int32)
    dst = edge_index[1].astype(jnp.int32)
    x = x.astype(jnp.float32)
    edge_attr = edge_attr.astype(jnp.float32)

    Wr = W_fc.reshape(D, H, D)
    alvec = jnp.einsum("dhk,hk->dh", Wr, attn_l[0])
    arvec = jnp.einsum("dhk,hk->dh", Wr, attn_r[0])
    wsmall = jnp.zeros((D, D), jnp.float32)
    wsmall = wsmall.at[:, 0:2].set(alvec).at[:, 2:4].set(arvec)

    nodes = pl.pallas_call(
        _nodes_body,
        out_shape=jax.ShapeDtypeStruct((N, D), jnp.float32),
    )(x, wsmall)

    BE = 2000
    eal8 = pl.pallas_call(
        _eal_body,
        grid=(E // BE,),
        in_specs=[pl.BlockSpec((BE, D), lambda i: (i, 0)),
                  pl.BlockSpec((D, D), lambda i: (0, 0))],
        out_specs=pl.BlockSpec((BE, 8), lambda i: (i, 0)),
        out_shape=jax.ShapeDtypeStruct((E, 8), jnp.float32),
    )(edge_attr, wsmall)

    xl_flat = nodes[:, 0:2].T.reshape(-1)
    er_flat = nodes[:, 2:4].T.reshape(-1)
    xlw = jnp.tile(xl_flat[:, None], (1, 16))
    erw = jnp.tile(er_flat[:, None], (1, 16))
    eal_flat = eal8[:, 0:2].T.reshape(-1)
    zeros_acc = jnp.zeros((N, WROW), jnp.float32)

    sc = pl.kernel(
        functools.partial(_sc_body),
        out_type=jax.ShapeDtypeStruct((2 * N, WROW), jnp.float32),
        mesh=plsc.VectorSubcoreMesh(core_axis_name="c", subcore_axis_name="s",
                                    num_cores=NC, num_subcores=NS),
        compiler_params=pltpu.CompilerParams(use_tc_tiling_on_sc=False,
                                             needs_layout_passes=False),
        scratch_types=[
            pltpu.VMEM_SHARED((N, WROW), jnp.float32),
            pltpu.VMEM((SB,), jnp.int32),
            pltpu.VMEM((2, CPS, C), jnp.int32),
            pltpu.VMEM((SB,), jnp.int32),
            pltpu.VMEM((SB,), jnp.int32),
            pltpu.VMEM((SB,), jnp.float32),
            pltpu.VMEM((16,), jnp.float32),
            pltpu.VMEM((SLOTS, C, D), jnp.float32),
            pltpu.VMEM((SLOTS, C, D), jnp.float32),
            pltpu.VMEM((SLOTS, C, WROW), jnp.float32),
            pltpu.VMEM((SLOTS, C, 16), jnp.float32),
            pltpu.VMEM((SLOTS, C, 16), jnp.float32),
        ] + [pltpu.SemaphoreType.DMA] * (5 * SLOTS),
    )
    dst2d = dst.reshape(CROWS, C)
    zu = sc(x, edge_attr, src, dst2d, eal_flat, xlw, erw, zeros_acc)

    g2 = gamma.reshape(1, D).astype(jnp.float32)
    b2 = beta.reshape(1, D).astype(jnp.float32)
    out = pl.pallas_call(
        _finish_body,
        out_shape=jax.ShapeDtypeStruct((N, D), jnp.float32),
    )(zu, W_fc.astype(jnp.float32), g2, b2)
    return out


# speedup vs baseline: 1.1802x; 1.1802x over previous
import functools

import numpy as np

import jax
import jax.numpy as jnp
from jax import lax
from jax.experimental import pallas as pl
from jax.experimental.pallas import tpu as pltpu
from jax.experimental.pallas import tpu_sc as plsc

N = 10000
E = 320000
D = 128
H = 2
NEG = 0.2
NS = 16
NC = 2
EPT = E // NS
C = 16
CPS = 50
SLOTS = 2
QPS = CPS // SLOTS
SB = C * CPS
NSB = EPT // SB
GPS = SB // 16
CG = C // 16
CROWS = E // C
RPT = N // NS
WROW = 144


def _nodes_body(x_ref, w_ref, xlw_ref, erw_ref):
    m = jnp.dot(x_ref[...], w_ref[...], preferred_element_type=jnp.float32)
    xlw_ref[0:N, :] = jnp.broadcast_to(m[:, 0:1], (N, 16))
    xlw_ref[N:2 * N, :] = jnp.broadcast_to(m[:, 1:2], (N, 16))
    erw_ref[0:N, :] = jnp.broadcast_to(m[:, 2:3], (N, 16))
    erw_ref[N:2 * N, :] = jnp.broadcast_to(m[:, 3:4], (N, 16))


def _eal_body(ea_ref, w_ref, o_ref):
    acc = jnp.dot(ea_ref[...], w_ref[...], preferred_element_type=jnp.float32)
    o_ref[...] = acc[:, :8]


def _finish_body(zu_ref, w_ref, g_ref, b_ref, o_ref):
    zu = zu_ref[...]
    w = w_ref[...]
    den0 = jnp.maximum(zu[:N, 128:129], 1e-38)
    den1 = jnp.maximum(zu[N:, 128:129], 1e-38)
    z0 = jnp.where(zu[:N, 128:129] > 0, zu[:N, :D] / den0, 0.0)
    z1 = jnp.where(zu[N:, 128:129] > 0, zu[N:, :D] / den1, 0.0)
    ft0 = jnp.dot(z0, w[:, :D], preferred_element_type=jnp.float32)
    ft1 = jnp.dot(z1, w[:, D:], preferred_element_type=jnp.float32)
    h = 0.5 * (ft0 + ft1)
    mu = jnp.mean(h, axis=0, keepdims=True)
    var = jnp.mean((h - mu) ** 2, axis=0, keepdims=True)
    hn = (h - mu) * lax.rsqrt(var + 1e-5) * g_ref[...] + b_ref[...]
    o_ref[...] = jnp.maximum(hn, 0.0)


def _sc_body(x_hbm, ea_hbm, src_hbm, dst2d_hbm, eal_hbm, xlw_hbm,
             erw_hbm, zeros_hbm,
             zu_out,
             zu_sh, sidx, didx2, sadj, dadj, ealb, p16,
             ea2, x2, out2, xlg2, erg2,
             *sems):
    c = lax.axis_index("c")
    s = lax.axis_index("s")
    cn = c * N
    sem_ea = sems[0:SLOTS]
    sem_x = sems[SLOTS:2 * SLOTS]
    sem_l = sems[2 * SLOTS:3 * SLOTS]
    sem_r = sems[3 * SLOTS:4 * SLOTS]
    sem_sc = sems[4 * SLOTS:5 * SLOTS]

    pltpu.sync_copy(zeros_hbm.at[pl.ds(s * RPT, RPT)],
                    zu_sh.at[pl.ds(s * RPT, RPT)])
    plsc.subcore_barrier()

    base = s * EPT
    rowbase = s * (EPT // C)

    def issue(k, j, b):
        e0g = base + k * SB + j * C
        off = j * C
        pltpu.async_copy(ea_hbm.at[pl.ds(e0g, C)], ea2.at[b], sem_ea[b])
        pltpu.async_copy(x_hbm.at[sidx.at[pl.ds(off, C)]], x2.at[b],
                         sem_x[b])
        pltpu.async_copy(xlw_hbm.at[sadj.at[pl.ds(off, C)]], xlg2.at[b],
                         sem_l[b])
        pltpu.async_copy(erw_hbm.at[dadj.at[pl.ds(off, C)]], erg2.at[b],
                         sem_r[b])

    def wait_loads(b):
        pltpu.make_async_copy(ea_hbm.at[pl.ds(0, C)], ea2.at[b],
                              sem_ea[b]).wait()
        pltpu.make_async_copy(x_hbm.at[pl.ds(0, C)], x2.at[b],
                              sem_x[b]).wait()
        pltpu.make_async_copy(xlw_hbm.at[pl.ds(0, C)], xlg2.at[b],
                              sem_l[b]).wait()
        pltpu.make_async_copy(erw_hbm.at[pl.ds(0, C)], erg2.at[b],
                              sem_r[b]).wait()

    def wait_scatter(b, kb):
        pltpu.make_async_copy(out2.at[b], zu_sh.at[didx2.at[kb, 0]],
                              sem_sc[b]).wait()

    def compute(j, b):
        rid = lax.iota(jnp.int32, 16)
        zc = jnp.zeros((16,), jnp.int32)
        xlv = plsc.load_gather(xlg2.at[b], [rid, zc])
        erv = plsc.load_gather(erg2.at[b], [rid, zc])
        ealv = plsc.load_gather(ealb, [j * C + rid, zc + c])
        att = xlv + ealv + erv
        att = jnp.where(att > 0, att, NEG * att)
        pv = jnp.exp(att)

        for jj in range(C):
            ps = pv[jj]
            for f in range(D // 16):
                out2[b, jj, pl.ds(f * 16, 16)] = (
                    x2[b, jj, pl.ds(f * 16, 16)]
                    + ea2[b, jj, pl.ds(f * 16, 16)]) * ps
            out2[b, jj, pl.ds(D, 16)] = jnp.full((16,), ps, jnp.float32)

    def scatter(j, b, kb):
        pltpu.async_copy(out2.at[b], zu_sh.at[didx2.at[kb, j]],
                         sem_sc[b], add=True)

    def sb_body(k, carry):
        kb = lax.rem(k, 2)
        e0 = base + k * SB
        pltpu.sync_copy(src_hbm.at[pl.ds(e0, SB)], sidx)
        pltpu.sync_copy(dst2d_hbm.at[pl.ds(rowbase + k * CPS, CPS)],
                        didx2.at[kb])
        pltpu.sync_copy(eal_hbm.at[pl.ds(e0, SB)], ealb)

        def adj_body(g, carry2):
            off = g * 16
            r = lax.div(g, CG)
            co = lax.rem(g, CG) * 16
            sadj[pl.ds(off, 16)] = sidx[pl.ds(off, 16)] + cn
            dadj[pl.ds(off, 16)] = didx2[kb, r, pl.ds(co, 16)] + cn
            return carry2

        lax.fori_loop(0, GPS, adj_body, 0)

        for b in range(SLOTS - 1):
            issue(k, b, b)

        def quint_body(q, carry2):
            j0 = SLOTS * q
            for i in range(SLOTS):
                j = j0 + i
                nb = (i + SLOTS - 1) % SLOTS

                @pl.when(j + SLOTS - 1 < CPS)
                def _(j=j, nb=nb):
                    issue(k, j + SLOTS - 1, nb)

                wait_loads(i)

                @pl.when(jnp.logical_or(k > 0, q >= 1))
                def _(i=i):
                    wait_scatter(i, kb)

                compute(j, i)
                scatter(j, i, kb)
            return carry2

        lax.fori_loop(0, QPS, quint_body, 0)
        return carry

    lax.fori_loop(0, NSB, sb_body, 0)
    for b in range(SLOTS):
        wait_scatter(b, 0)
    plsc.subcore_barrier()
    pltpu.sync_copy(zu_sh.at[pl.ds(s * RPT, RPT)],
                    zu_out.at[pl.ds(c * N + s * RPT, RPT)])


def kernel(x, edge_index, edge_attr, W_fc, attn_l, attn_r, gamma, beta):
    src = edge_index[0].astype(jnp.int32)
    dst = edge_index[1].astype(jnp.int32)
    x = x.astype(jnp.float32)
    edge_attr = edge_attr.astype(jnp.float32)

    Wr = W_fc.reshape(D, H, D)
    alvec = jnp.einsum("dhk,hk->dh", Wr, attn_l[0])
    arvec = jnp.einsum("dhk,hk->dh", Wr, attn_r[0])
    wsmall = jnp.zeros((D, D), jnp.float32)
    wsmall = wsmall.at[:, 0:2].set(alvec).at[:, 2:4].set(arvec)

    xlw, erw = pl.pallas_call(
        _nodes_body,
        out_shape=[jax.ShapeDtypeStruct((2 * N, 16), jnp.float32),
                   jax.ShapeDtypeStruct((2 * N, 16), jnp.float32)],
    )(x, wsmall)

    BE = 2000
    eal8 = pl.pallas_call(
        _eal_body,
        grid=(E // BE,),
        in_specs=[pl.BlockSpec((BE, D), lambda i: (i, 0)),
                  pl.BlockSpec((D, D), lambda i: (0, 0))],
        out_specs=pl.BlockSpec((BE, 8), lambda i: (i, 0)),
        out_shape=jax.ShapeDtypeStruct((E, 8), jnp.float32),
    )(edge_attr, wsmall)

    zeros_acc = jnp.zeros((N, WROW), jnp.float32)

    sc = pl.kernel(
        functools.partial(_sc_body),
        out_type=jax.ShapeDtypeStruct((2 * N, WROW), jnp.float32),
        mesh=plsc.VectorSubcoreMesh(core_axis_name="c", subcore_axis_name="s",
                                    num_cores=NC, num_subcores=NS),
        compiler_params=pltpu.CompilerParams(use_tc_tiling_on_sc=False,
                                             needs_layout_passes=False),
        scratch_types=[
            pltpu.VMEM_SHARED((N, WROW), jnp.float32),
            pltpu.VMEM((SB,), jnp.int32),
            pltpu.VMEM((2, CPS, C), jnp.int32),
            pltpu.VMEM((SB,), jnp.int32),
            pltpu.VMEM((SB,), jnp.int32),
            pltpu.VMEM((SB, 8), jnp.float32),
            pltpu.VMEM((16,), jnp.float32),
            pltpu.VMEM((SLOTS, C, D), jnp.float32),
            pltpu.VMEM((SLOTS, C, D), jnp.float32),
            pltpu.VMEM((SLOTS, C, WROW), jnp.float32),
            pltpu.VMEM((SLOTS, C, 16), jnp.float32),
            pltpu.VMEM((SLOTS, C, 16), jnp.float32),
        ] + [pltpu.SemaphoreType.DMA] * (5 * SLOTS),
    )
    dst2d = dst.reshape(CROWS, C)
    zu = sc(x, edge_attr, src, dst2d, eal8, xlw, erw, zeros_acc)

    g2 = gamma.reshape(1, D).astype(jnp.float32)
    b2 = beta.reshape(1, D).astype(jnp.float32)
    out = pl.pallas_call(
        _finish_body,
        out_shape=jax.ShapeDtypeStruct((N, D), jnp.float32),
    )(zu, W_fc.astype(jnp.float32), g2, b2)
    return out


# speedup vs baseline: 1.2080x; 1.0236x over previous
import functools

import numpy as np

import jax
import jax.numpy as jnp
from jax import lax
from jax.experimental import pallas as pl
from jax.experimental.pallas import tpu as pltpu
from jax.experimental.pallas import tpu_sc as plsc

N = 10000
E = 320000
D = 128
H = 2
NEG = 0.2
NS = 16
NC = 2
EPT = E // NS
C = 16
CPS = 50
SLOTS = 2
QPS = CPS // SLOTS
SB = C * CPS
NSB = EPT // SB
GPS = SB // 16
CG = C // 16
CROWS = E // C
RPT = N // NS
WROW = 144


def _nodes_body(x_ref, w_ref, xlw_ref, erw_ref):
    m = jnp.dot(x_ref[...], w_ref[...], preferred_element_type=jnp.float32)
    xlw_ref[0:N, :] = jnp.broadcast_to(m[:, 0:1], (N, 16))
    xlw_ref[N:2 * N, :] = jnp.broadcast_to(m[:, 1:2], (N, 16))
    erw_ref[0:N, :] = jnp.broadcast_to(m[:, 2:3], (N, 16))
    erw_ref[N:2 * N, :] = jnp.broadcast_to(m[:, 3:4], (N, 16))


def _eal_body(ea_ref, w_ref, o_ref):
    acc = jnp.dot(ea_ref[...], w_ref[...], preferred_element_type=jnp.float32)
    o_ref[...] = acc[:, :8]


def _finish_body(zu_ref, w_ref, g_ref, b_ref, o_ref):
    zu = zu_ref[...]
    w = w_ref[...]
    den0 = jnp.maximum(zu[:N, 128:129], 1e-38)
    den1 = jnp.maximum(zu[N:, 128:129], 1e-38)
    z0 = jnp.where(zu[:N, 128:129] > 0, zu[:N, :D] / den0, 0.0)
    z1 = jnp.where(zu[N:, 128:129] > 0, zu[N:, :D] / den1, 0.0)
    ft0 = jnp.dot(z0, w[:, :D], preferred_element_type=jnp.float32)
    ft1 = jnp.dot(z1, w[:, D:], preferred_element_type=jnp.float32)
    h = 0.5 * (ft0 + ft1)
    mu = jnp.mean(h, axis=0, keepdims=True)
    var = jnp.mean((h - mu) ** 2, axis=0, keepdims=True)
    hn = (h - mu) * lax.rsqrt(var + 1e-5) * g_ref[...] + b_ref[...]
    o_ref[...] = jnp.maximum(hn, 0.0)


def _sc_body(x_hbm, ea_hbm, src_hbm, dst2d_hbm, eal_hbm, xlw_hbm,
             erw_hbm, zeros_hbm,
             zu_out,
             zu_sh, sidx, didx2, sadj, dadj, ealb, p16,
             ea2, x2, out2, xlg2, erg2,
             *sems):
    c = lax.axis_index("c")
    s = lax.axis_index("s")
    cn = c * N
    sem_ea = sems[0:SLOTS]
    sem_x = sems[SLOTS:2 * SLOTS]
    sem_l = sems[2 * SLOTS:3 * SLOTS]
    sem_r = sems[3 * SLOTS:4 * SLOTS]
    sem_sc = sems[4 * SLOTS:5 * SLOTS]
    sem_idx = sems[5 * SLOTS]

    pltpu.sync_copy(zeros_hbm.at[pl.ds(s * RPT, RPT)],
                    zu_sh.at[pl.ds(s * RPT, RPT)])
    plsc.subcore_barrier()

    base = s * EPT
    rowbase = s * (EPT // C)

    def issue(k, j, b):
        e0g = base + k * SB + j * C
        off = j * C
        pltpu.async_copy(ea_hbm.at[pl.ds(e0g, C)], ea2.at[b], sem_ea[b])
        pltpu.async_copy(x_hbm.at[sidx.at[pl.ds(off, C)]], x2.at[b],
                         sem_x[b])
        pltpu.async_copy(xlw_hbm.at[sadj.at[pl.ds(off, C)]], xlg2.at[b],
                         sem_l[b])
        pltpu.async_copy(erw_hbm.at[dadj.at[pl.ds(off, C)]], erg2.at[b],
                         sem_r[b])

    def wait_loads(b):
        pltpu.make_async_copy(ea_hbm.at[pl.ds(0, C)], ea2.at[b],
                              sem_ea[b]).wait()
        pltpu.make_async_copy(x_hbm.at[pl.ds(0, C)], x2.at[b],
                              sem_x[b]).wait()
        pltpu.make_async_copy(xlw_hbm.at[pl.ds(0, C)], xlg2.at[b],
                              sem_l[b]).wait()
        pltpu.make_async_copy(erw_hbm.at[pl.ds(0, C)], erg2.at[b],
                              sem_r[b]).wait()

    def wait_scatter(b, kb):
        pltpu.make_async_copy(out2.at[b], zu_sh.at[didx2.at[kb, 0]],
                              sem_sc[b]).wait()

    def compute(j, b):
        rid = lax.iota(jnp.int32, 16)
        zc = jnp.zeros((16,), jnp.int32)
        xlv = plsc.load_gather(xlg2.at[b], [rid, zc])
        erv = plsc.load_gather(erg2.at[b], [rid, zc])
        ealv = plsc.load_gather(ealb, [j * C + rid, zc + c])
        att = xlv + ealv + erv
        att = jnp.where(att > 0, att, NEG * att)
        pv = jnp.exp(att)

        for jj in range(C):
            ps = pv[jj]
            for f in range(D // 16):
                out2[b, jj, pl.ds(f * 16, 16)] = (
                    x2[b, jj, pl.ds(f * 16, 16)]
                    + ea2[b, jj, pl.ds(f * 16, 16)]) * ps
            out2[b, jj, pl.ds(D, 16)] = jnp.full((16,), ps, jnp.float32)

    def scatter(j, b, kb):
        pltpu.async_copy(out2.at[b], zu_sh.at[didx2.at[kb, j]],
                         sem_sc[b], add=True)

    def sb_body(k, carry):
        kb = lax.rem(k, 2)
        e0 = base + k * SB
        pltpu.async_copy(src_hbm.at[pl.ds(e0, SB)], sidx, sem_idx)
        pltpu.async_copy(dst2d_hbm.at[pl.ds(rowbase + k * CPS, CPS)],
                         didx2.at[kb], sem_idx)
        pltpu.async_copy(eal_hbm.at[pl.ds(e0, SB)], ealb, sem_idx)
        pltpu.make_async_copy(src_hbm.at[pl.ds(0, SB)], sidx,
                              sem_idx).wait()
        pltpu.make_async_copy(dst2d_hbm.at[pl.ds(0, CPS)], didx2.at[kb],
                              sem_idx).wait()
        pltpu.make_async_copy(eal_hbm.at[pl.ds(0, SB)], ealb,
                              sem_idx).wait()

        def adj_body(g, carry2):
            off = g * 16
            r = lax.div(g, CG)
            co = lax.rem(g, CG) * 16
            sadj[pl.ds(off, 16)] = sidx[pl.ds(off, 16)] + cn
            dadj[pl.ds(off, 16)] = didx2[kb, r, pl.ds(co, 16)] + cn
            return carry2

        lax.fori_loop(0, GPS, adj_body, 0)

        for b in range(SLOTS - 1):
            issue(k, b, b)

        def quint_body(q, carry2):
            j0 = SLOTS * q
            for i in range(SLOTS):
                j = j0 + i
                nb = (i + SLOTS - 1) % SLOTS

                @pl.when(j + SLOTS - 1 < CPS)
                def _(j=j, nb=nb):
                    issue(k, j + SLOTS - 1, nb)

                wait_loads(i)

                @pl.when(jnp.logical_or(k > 0, q >= 1))
                def _(i=i):
                    wait_scatter(i, kb)

                compute(j, i)
                scatter(j, i, kb)
            return carry2

        lax.fori_loop(0, QPS, quint_body, 0)
        return carry

    lax.fori_loop(0, NSB, sb_body, 0)
    for b in range(SLOTS):
        wait_scatter(b, 0)
    plsc.subcore_barrier()
    pltpu.sync_copy(zu_sh.at[pl.ds(s * RPT, RPT)],
                    zu_out.at[pl.ds(c * N + s * RPT, RPT)])


def kernel(x, edge_index, edge_attr, W_fc, attn_l, attn_r, gamma, beta):
    src = edge_index[0].astype(jnp.int32)
    dst = edge_index[1].astype(jnp.int32)
    x = x.astype(jnp.float32)
    edge_attr = edge_attr.astype(jnp.float32)

    Wr = W_fc.reshape(D, H, D)
    alvec = jnp.einsum("dhk,hk->dh", Wr, attn_l[0])
    arvec = jnp.einsum("dhk,hk->dh", Wr, attn_r[0])
    wsmall = jnp.zeros((D, D), jnp.float32)
    wsmall = wsmall.at[:, 0:2].set(alvec).at[:, 2:4].set(arvec)

    xlw, erw = pl.pallas_call(
        _nodes_body,
        out_shape=[jax.ShapeDtypeStruct((2 * N, 16), jnp.float32),
                   jax.ShapeDtypeStruct((2 * N, 16), jnp.float32)],
    )(x, wsmall)

    BE = 2000
    eal8 = pl.pallas_call(
        _eal_body,
        grid=(E // BE,),
        in_specs=[pl.BlockSpec((BE, D), lambda i: (i, 0)),
                  pl.BlockSpec((D, D), lambda i: (0, 0))],
        out_specs=pl.BlockSpec((BE, 8), lambda i: (i, 0)),
        out_shape=jax.ShapeDtypeStruct((E, 8), jnp.float32),
    )(edge_attr, wsmall)

    zeros_acc = jnp.zeros((N, WROW), jnp.float32)

    sc = pl.kernel(
        functools.partial(_sc_body),
        out_type=jax.ShapeDtypeStruct((2 * N, WROW), jnp.float32),
        mesh=plsc.VectorSubcoreMesh(core_axis_name="c", subcore_axis_name="s",
                                    num_cores=NC, num_subcores=NS),
        compiler_params=pltpu.CompilerParams(use_tc_tiling_on_sc=False,
                                             needs_layout_passes=False),
        scratch_types=[
            pltpu.VMEM_SHARED((N, WROW), jnp.float32),
            pltpu.VMEM((SB,), jnp.int32),
            pltpu.VMEM((2, CPS, C), jnp.int32),
            pltpu.VMEM((SB,), jnp.int32),
            pltpu.VMEM((SB,), jnp.int32),
            pltpu.VMEM((SB, 8), jnp.float32),
            pltpu.VMEM((16,), jnp.float32),
            pltpu.VMEM((SLOTS, C, D), jnp.float32),
            pltpu.VMEM((SLOTS, C, D), jnp.float32),
            pltpu.VMEM((SLOTS, C, WROW), jnp.float32),
            pltpu.VMEM((SLOTS, C, 16), jnp.float32),
            pltpu.VMEM((SLOTS, C, 16), jnp.float32),
        ] + [pltpu.SemaphoreType.DMA] * (5 * SLOTS + 1),
    )
    dst2d = dst.reshape(CROWS, C)
    zu = sc(x, edge_attr, src, dst2d, eal8, xlw, erw, zeros_acc)

    g2 = gamma.reshape(1, D).astype(jnp.float32)
    b2 = beta.reshape(1, D).astype(jnp.float32)
    out = pl.pallas_call(
        _finish_body,
        out_shape=jax.ShapeDtypeStruct((N, D), jnp.float32),
    )(zu, W_fc.astype(jnp.float32), g2, b2)
    return out


# speedup vs baseline: 1.2081x; 1.0001x over previous
import functools

import jax
import jax.numpy as jnp
from jax import lax
from jax.experimental import pallas as pl
from jax.experimental.pallas import tpu as pltpu
from jax.experimental.pallas import tpu_sc as plsc

N = 10000
E = 320000
D = 128
H = 2
NEG = 0.2
NS = 16
NC = 2
EPT = E // NS
C = 16
CPS = 50
SLOTS = 2
QPS = CPS // SLOTS
SB = C * CPS
NSB = EPT // SB
GPS = SB // 16
CG = C // 16
CROWS = E // C
RPT = N // NS
WROW = 144


def _nodes_body(x_ref, w_ref, xlw_ref, erw_ref):
    m = jnp.dot(x_ref[...], w_ref[...], preferred_element_type=jnp.float32)
    xlw_ref[0:N, :] = jnp.broadcast_to(m[:, 0:1], (N, 16))
    xlw_ref[N:2 * N, :] = jnp.broadcast_to(m[:, 1:2], (N, 16))
    erw_ref[0:N, :] = jnp.broadcast_to(m[:, 2:3], (N, 16))
    erw_ref[N:2 * N, :] = jnp.broadcast_to(m[:, 3:4], (N, 16))


def _eal_body(ea_ref, w_ref, o_ref):
    acc = jnp.dot(ea_ref[...], w_ref[...], preferred_element_type=jnp.float32)
    o_ref[...] = acc[:, :8]


def _finish_body(zu_ref, w_ref, g_ref, b_ref, o_ref):
    zu = zu_ref[...]
    w = w_ref[...]
    den0 = jnp.maximum(zu[:N, 128:129], 1e-38)
    den1 = jnp.maximum(zu[N:, 128:129], 1e-38)
    z0 = jnp.where(zu[:N, 128:129] > 0, zu[:N, :D] / den0, 0.0)
    z1 = jnp.where(zu[N:, 128:129] > 0, zu[N:, :D] / den1, 0.0)
    ft0 = jnp.dot(z0, w[:, :D], preferred_element_type=jnp.float32)
    ft1 = jnp.dot(z1, w[:, D:], preferred_element_type=jnp.float32)
    h = 0.5 * (ft0 + ft1)
    mu = jnp.mean(h, axis=0, keepdims=True)
    var = jnp.mean((h - mu) ** 2, axis=0, keepdims=True)
    hn = (h - mu) * lax.rsqrt(var + 1e-5) * g_ref[...] + b_ref[...]
    o_ref[...] = jnp.maximum(hn, 0.0)


def _sc_body(x_hbm, ea_hbm, src_hbm, dst2d_hbm, eal_hbm, xlw_hbm,
             erw_hbm, zeros_hbm,
             zu_out,
             zu_sh, sidx, didx2, sadj, dadj, ealb,
             ea2, x2, out2, xlg2, erg2,
             *sems):
    c = lax.axis_index("c")
    s = lax.axis_index("s")
    cn = c * N
    sem_ea = sems[0:SLOTS]
    sem_x = sems[SLOTS:2 * SLOTS]
    sem_l = sems[2 * SLOTS:3 * SLOTS]
    sem_r = sems[3 * SLOTS:4 * SLOTS]
    sem_sc = sems[4 * SLOTS:5 * SLOTS]
    sem_idx = sems[5 * SLOTS]

    pltpu.sync_copy(zeros_hbm.at[pl.ds(s * RPT, RPT)],
                    zu_sh.at[pl.ds(s * RPT, RPT)])
    plsc.subcore_barrier()

    base = s * EPT
    rowbase = s * (EPT // C)

    def issue(k, j, b):
        e0g = base + k * SB + j * C
        off = j * C
        pltpu.async_copy(ea_hbm.at[pl.ds(e0g, C)], ea2.at[b], sem_ea[b])
        pltpu.async_copy(x_hbm.at[sidx.at[pl.ds(off, C)]], x2.at[b],
                         sem_x[b])
        pltpu.async_copy(xlw_hbm.at[sadj.at[pl.ds(off, C)]], xlg2.at[b],
                         sem_l[b])
        pltpu.async_copy(erw_hbm.at[dadj.at[pl.ds(off, C)]], erg2.at[b],
                         sem_r[b])

    def wait_loads(b):
        pltpu.make_async_copy(ea_hbm.at[pl.ds(0, C)], ea2.at[b],
                              sem_ea[b]).wait()
        pltpu.make_async_copy(x_hbm.at[pl.ds(0, C)], x2.at[b],
                              sem_x[b]).wait()
        pltpu.make_async_copy(xlw_hbm.at[pl.ds(0, C)], xlg2.at[b],
                              sem_l[b]).wait()
        pltpu.make_async_copy(erw_hbm.at[pl.ds(0, C)], erg2.at[b],
                              sem_r[b]).wait()

    def wait_scatter(b, kb):
        pltpu.make_async_copy(out2.at[b], zu_sh.at[didx2.at[kb, 0]],
                              sem_sc[b]).wait()

    def compute(j, b):
        rid = lax.iota(jnp.int32, 16)
        zc = jnp.zeros((16,), jnp.int32)
        xlv = plsc.load_gather(xlg2.at[b], [rid, zc])
        erv = plsc.load_gather(erg2.at[b], [rid, zc])
        ealv = plsc.load_gather(ealb, [j * C + rid, zc + c])
        att = xlv + ealv + erv
        att = jnp.where(att > 0, att, NEG * att)
        pv = jnp.exp(att)

        for jj in range(C):
            ps = pv[jj]
            for f in range(D // 16):
                out2[b, jj, pl.ds(f * 16, 16)] = (
                    x2[b, jj, pl.ds(f * 16, 16)]
                    + ea2[b, jj, pl.ds(f * 16, 16)]) * ps
            out2[b, jj, pl.ds(D, 16)] = jnp.full((16,), ps, jnp.float32)

    def scatter(j, b, kb):
        pltpu.async_copy(out2.at[b], zu_sh.at[didx2.at[kb, j]],
                         sem_sc[b], add=True)

    def sb_body(k, carry):
        kb = lax.rem(k, 2)
        e0 = base + k * SB
        pltpu.async_copy(src_hbm.at[pl.ds(e0, SB)], sidx, sem_idx)
        pltpu.async_copy(dst2d_hbm.at[pl.ds(rowbase + k * CPS, CPS)],
                         didx2.at[kb], sem_idx)
        pltpu.async_copy(eal_hbm.at[pl.ds(e0, SB)], ealb, sem_idx)
        pltpu.make_async_copy(src_hbm.at[pl.ds(0, SB)], sidx,
                              sem_idx).wait()
        pltpu.make_async_copy(dst2d_hbm.at[pl.ds(0, CPS)], didx2.at[kb],
                              sem_idx).wait()
        pltpu.make_async_copy(eal_hbm.at[pl.ds(0, SB)], ealb,
                              sem_idx).wait()

        def adj_body(g, carry2):
            off = g * 16
            r = lax.div(g, CG)
            co = lax.rem(g, CG) * 16
            sadj[pl.ds(off, 16)] = sidx[pl.ds(off, 16)] + cn
            dadj[pl.ds(off, 16)] = didx2[kb, r, pl.ds(co, 16)] + cn
            return carry2

        lax.fori_loop(0, GPS, adj_body, 0)

        for b in range(SLOTS - 1):
            issue(k, b, b)

        def quint_body(q, carry2):
            j0 = SLOTS * q
            for i in range(SLOTS):
                j = j0 + i
                nb = (i + SLOTS - 1) % SLOTS

                @pl.when(j + SLOTS - 1 < CPS)
                def _(j=j, nb=nb):
                    issue(k, j + SLOTS - 1, nb)

                wait_loads(i)

                @pl.when(jnp.logical_or(k > 0, q >= 1))
                def _(i=i):
                    wait_scatter(i, kb)

                compute(j, i)
                scatter(j, i, kb)
            return carry2

        lax.fori_loop(0, QPS, quint_body, 0)
        return carry

    lax.fori_loop(0, NSB, sb_body, 0)
    for b in range(SLOTS):
        wait_scatter(b, 0)
    plsc.subcore_barrier()
    pltpu.sync_copy(zu_sh.at[pl.ds(s * RPT, RPT)],
                    zu_out.at[pl.ds(c * N + s * RPT, RPT)])


def kernel(x, edge_index, edge_attr, W_fc, attn_l, attn_r, gamma, beta):
    src = edge_index[0].astype(jnp.int32)
    dst = edge_index[1].astype(jnp.int32)
    x = x.astype(jnp.float32)
    edge_attr = edge_attr.astype(jnp.float32)

    Wr = W_fc.reshape(D, H, D)
    alvec = jnp.einsum("dhk,hk->dh", Wr, attn_l[0])
    arvec = jnp.einsum("dhk,hk->dh", Wr, attn_r[0])
    wsmall = jnp.zeros((D, D), jnp.float32)
    wsmall = wsmall.at[:, 0:2].set(alvec).at[:, 2:4].set(arvec)

    xlw, erw = pl.pallas_call(
        _nodes_body,
        out_shape=[jax.ShapeDtypeStruct((2 * N, 16), jnp.float32),
                   jax.ShapeDtypeStruct((2 * N, 16), jnp.float32)],
    )(x, wsmall)

    BE = 2000
    eal8 = pl.pallas_call(
        _eal_body,
        grid=(E // BE,),
        in_specs=[pl.BlockSpec((BE, D), lambda i: (i, 0)),
                  pl.BlockSpec((D, D), lambda i: (0, 0))],
        out_specs=pl.BlockSpec((BE, 8), lambda i: (i, 0)),
        out_shape=jax.ShapeDtypeStruct((E, 8), jnp.float32),
    )(edge_attr, wsmall)

    zeros_acc = jnp.zeros((N, WROW), jnp.float32)

    sc = pl.kernel(
        functools.partial(_sc_body),
        out_type=jax.ShapeDtypeStruct((2 * N, WROW), jnp.float32),
        mesh=plsc.VectorSubcoreMesh(core_axis_name="c", subcore_axis_name="s",
                                    num_cores=NC, num_subcores=NS),
        compiler_params=pltpu.CompilerParams(use_tc_tiling_on_sc=False,
                                             needs_layout_passes=False),
        scratch_types=[
            pltpu.VMEM_SHARED((N, WROW), jnp.float32),
            pltpu.VMEM((SB,), jnp.int32),
            pltpu.VMEM((2, CPS, C), jnp.int32),
            pltpu.VMEM((SB,), jnp.int32),
            pltpu.VMEM((SB,), jnp.int32),
            pltpu.VMEM((SB, 8), jnp.float32),
            pltpu.VMEM((SLOTS, C, D), jnp.float32),
            pltpu.VMEM((SLOTS, C, D), jnp.float32),
            pltpu.VMEM((SLOTS, C, WROW), jnp.float32),
            pltpu.VMEM((SLOTS, C, 16), jnp.float32),
            pltpu.VMEM((SLOTS, C, 16), jnp.float32),
        ] + [pltpu.SemaphoreType.DMA] * (5 * SLOTS + 1),
    )
    dst2d = dst.reshape(CROWS, C)
    zu = sc(x, edge_attr, src, dst2d, eal8, xlw, erw, zeros_acc)

    g2 = gamma.reshape(1, D).astype(jnp.float32)
    b2 = beta.reshape(1, D).astype(jnp.float32)
    out = pl.pallas_call(
        _finish_body,
        out_shape=jax.ShapeDtypeStruct((N, D), jnp.float32),
    )(zu, W_fc.astype(jnp.float32), g2, b2)
    return out
